# Initial kernel scaffold; baseline (speedup 1.0000x reference)
import jax
import jax.numpy as jnp
from jax.experimental import pallas as pl


def kernel(x, edge_index, Ws0, Wn0, b0, Ws1, Wn1, b1, Ws2, Wn2, b2, gamma, beta, Dw1, Db1, Dw2, Db2, Dw3, Db3):
    raise NotImplementedError("write your pallas kernel here")


# speedup vs baseline: 3.0648x; 3.0648x over previous
import functools

import jax
import jax.numpy as jnp
from jax import lax
from jax.experimental import pallas as pl
from jax.experimental.pallas import tpu as pltpu
from jax.experimental.pallas import tpu_sc as plsc

N = 10000
E = 320000
D = 128
NC, NS = 2, 16
C = 128
CH = 158
E_PAD = NS * CH * C
SPLIT = 5056
A_ROWS = 5120
RPT = A_ROWS // NS
NBUF = 2


def _make_agg(with_deg):
  mesh = plsc.VectorSubcoreMesh(core_axis_name="c", subcore_axis_name="s",
                                num_cores=NC)
  out_type = jax.ShapeDtypeStruct((NC, A_ROWS, D), jnp.float32)
  if with_deg:
    out_type = (out_type, jax.ShapeDtypeStruct((NC, NS, A_ROWS), jnp.float32))
  scratch = [
      pltpu.VMEM((CH, C), jnp.int32),
      pltpu.VMEM((CH, C), jnp.int32),
      pltpu.VMEM((NBUF, C, D), jnp.float32),
      pltpu.VMEM_SHARED((A_ROWS, D), jnp.float32),
      pltpu.SemaphoreType.DMA((NBUF,)),
      pltpu.SemaphoreType.DMA,
  ]
  if with_deg:
    scratch.append(pltpu.VMEM((A_ROWS,), jnp.float32))

  @functools.partial(
      pl.kernel, out_type=out_type, mesh=mesh, scratch_types=scratch,
      compiler_params=pltpu.CompilerParams(needs_layout_passes=False))
  def agg(h_hbm, src_hbm, dst_hbm, *rest):
    if with_deg:
      out_hbm, deg_hbm, src_v, dst_v, rows_v, acc, gsem, ssem, deg_v = rest
    else:
      out_hbm, src_v, dst_v, rows_v, acc, gsem, ssem = rest
    cid = lax.axis_index("c")
    sid = lax.axis_index("s")
    row0 = cid * SPLIT

    zero16 = jnp.zeros((16,), jnp.float32)
    nzv = D // 16

    def zrow(i, _):
      rows_v[0, i // nzv, pl.ds((i % nzv) * 16, 16)] = zero16
      return 0

    lax.fori_loop(0, C * nzv, zrow, 0)
    if with_deg:
      def zdeg(i, _):
        deg_v[pl.ds(i * 16, 16)] = zero16
        return 0
      lax.fori_loop(0, A_ROWS // 16, zdeg, 0)
    base = sid * RPT
    for k in range(RPT // C):
      pltpu.sync_copy(rows_v.at[0], acc.at[pl.ds(base + k * C, C)])
    rem = RPT - (RPT // C) * C
    if rem:
      pltpu.sync_copy(rows_v.at[0, pl.ds(0, rem)],
                      acc.at[pl.ds(base + (RPT // C) * C, rem)])

    pltpu.sync_copy(src_hbm.at[sid], src_v)
    pltpu.sync_copy(dst_hbm.at[sid], dst_v)

    trash16 = jnp.full((16,), SPLIT, jnp.int32)

    def remap(i, _):
      c = i // (C // 16)
      off = (i % (C // 16)) * 16
      v = dst_v[c, pl.ds(off, 16)] - row0
      oob = (v < 0) | (v >= SPLIT)
      dst_v[c, pl.ds(off, 16)] = jnp.where(oob, trash16, v)
      return 0

    lax.fori_loop(0, CH * (C // 16), remap, 0)
    plsc.subcore_barrier()

    for b in range(NBUF):
      pltpu.async_copy(h_hbm.at[src_v.at[b]], rows_v.at[b], gsem.at[b])

    ones16 = jnp.ones((16,), jnp.float32)

    def body(c, _):
      p = lax.rem(c, NBUF)
      pltpu.make_async_copy(h_hbm.at[src_v.at[c]], rows_v.at[p],
                            gsem.at[p]).wait()
      sc = pltpu.make_async_copy(rows_v.at[p], acc.at[dst_v.at[c]], ssem)
      sc.start(add=True)
      if with_deg:
        for j in range(C // 16):
          dvec = dst_v[c, pl.ds(j * 16, 16)]
          plsc.addupdate_scatter(deg_v, [dvec], ones16)
      sc.wait()

      @pl.when(c + NBUF < CH)
      def _():
        pltpu.async_copy(h_hbm.at[src_v.at[c + NBUF]], rows_v.at[p],
                         gsem.at[p])
      return 0

    lax.fori_loop(0, CH, body, 0)
    if with_deg:
      pltpu.sync_copy(deg_v, deg_hbm.at[cid, sid])
    plsc.subcore_barrier()

    pltpu.sync_copy(acc.at[pl.ds(base, RPT)],
                    out_hbm.at[cid, pl.ds(base, RPT)])

  return agg


_agg_l0 = _make_agg(True)
_agg = _make_agg(False)


def _assemble(p_ref):
  return jnp.concatenate([p_ref[0, :SPLIT, :], p_ref[1, :N - SPLIT, :]],
                         axis=0)


def _dense0_body(x_ref, p_ref, dh_ref, ws_ref, wn_ref, b_ref, g_ref, be_ref,
                 h_out, dinv_out):
  agg = _assemble(p_ref)
  ones = jnp.ones((NS, 1), jnp.float32)
  dn = (((0,), (0,)), ((), ()))
  d_lo = lax.dot_general(dh_ref[0], ones, dimension_numbers=dn,
                         preferred_element_type=jnp.float32)
  d_hi = lax.dot_general(dh_ref[1], ones, dimension_numbers=dn,
                         preferred_element_type=jnp.float32)
  deg = jnp.concatenate([d_lo[:SPLIT], d_hi[:N - SPLIT]], axis=0)
  dinv = 1.0 / jnp.maximum(deg, 1.0)
  hn = agg * dinv
  hpre = (jnp.dot(x_ref[...], ws_ref[...], preferred_element_type=jnp.float32)
          + jnp.dot(hn, wn_ref[...], preferred_element_type=jnp.float32)
          + b_ref[...])
  mu = jnp.mean(hpre, axis=0, keepdims=True)
  var = jnp.mean((hpre - mu) ** 2, axis=0, keepdims=True)
  y = (hpre - mu) * lax.rsqrt(var + 1e-5) * g_ref[...] + be_ref[...]
  h_out[...] = jnp.maximum(y, 0.0)
  dinv_out[...] = jnp.broadcast_to(dinv, (N, D))


def _dense1_body(h_ref, p_ref, dinv_ref, ws_ref, wn_ref, b_ref, g_ref, be_ref,
                 h_out):
  hn = _assemble(p_ref) * dinv_ref[...]
  hpre = (jnp.dot(h_ref[...], ws_ref[...], preferred_element_type=jnp.float32)
          + jnp.dot(hn, wn_ref[...], preferred_element_type=jnp.float32)
          + b_ref[...])
  mu = jnp.mean(hpre, axis=0, keepdims=True)
  var = jnp.mean((hpre - mu) ** 2, axis=0, keepdims=True)
  y = (hpre - mu) * lax.rsqrt(var + 1e-5) * g_ref[...] + be_ref[...]
  h_out[...] = jnp.maximum(y, 0.0)


def _dense2_body(h_ref, p_ref, dinv_ref, ws_ref, wn_ref, b_ref,
                 dw1_ref, db1_ref, dw2_ref, db2_ref, dw3_ref, db3_ref,
                 out_ref):
  hn = _assemble(p_ref) * dinv_ref[...]
  h = (jnp.dot(h_ref[...], ws_ref[...], preferred_element_type=jnp.float32)
       + jnp.dot(hn, wn_ref[...], preferred_element_type=jnp.float32)
       + b_ref[...])
  h = jnp.maximum(
      jnp.dot(h, dw1_ref[...], preferred_element_type=jnp.float32)
      + db1_ref[...], 0.0)
  h = jnp.maximum(
      jnp.dot(h, dw2_ref[...], preferred_element_type=jnp.float32)
      + db2_ref[...], 0.0)
  out_ref[...] = (jnp.dot(h, dw3_ref[...], preferred_element_type=jnp.float32)
                  + db3_ref[...])


def kernel(x, edge_index, Ws0, Wn0, b0, Ws1, Wn1, b1, Ws2, Wn2, b2,
           gamma, beta, Dw1, Db1, Dw2, Db2, Dw3, Db3):
  src = edge_index[0]
  dst = edge_index[1]
  pad = E_PAD - E
  src_p = jnp.concatenate([src, jnp.zeros((pad,), jnp.int32)]).reshape(NS, CH, C)
  dst_p = jnp.concatenate([dst, jnp.full((pad,), N, jnp.int32)]).reshape(NS, CH, C)

  b0_, b1_, b2_ = b0.reshape(1, D), b1.reshape(1, D), b2.reshape(1, D)
  g_, be_ = gamma.reshape(1, D), beta.reshape(1, D)
  Db1_, Db2_, Db3_ = Db1.reshape(1, D), Db2.reshape(1, D), Db3.reshape(1, 1)

  p0, dh = _agg_l0(x, src_p, dst_p)
  h1, dinv = pl.pallas_call(
      _dense0_body,
      out_shape=(jax.ShapeDtypeStruct((N, D), jnp.float32),
                 jax.ShapeDtypeStruct((N, D), jnp.float32)),
  )(x, p0, dh, Ws0, Wn0, b0_, g_, be_)

  p1 = _agg(h1, src_p, dst_p)
  h2 = pl.pallas_call(
      _dense1_body,
      out_shape=jax.ShapeDtypeStruct((N, D), jnp.float32),
  )(h1, p1, dinv, Ws1, Wn1, b1_, g_, be_)

  p2 = _agg(h2, src_p, dst_p)
  out = pl.pallas_call(
      _dense2_body,
      out_shape=jax.ShapeDtypeStruct((N, 1), jnp.float32),
  )(h2, p2, dinv, Ws2, Wn2, b2_, Dw1, Db1_, Dw2, Db2_, Dw3, Db3_)
  return out
